# baseline (speedup 1.0000x reference)
import functools
import jax
import jax.numpy as jnp
from jax import lax
from jax.experimental import pallas as pl
from jax.experimental.pallas import tpu as pltpu
from jax.experimental.pallas import tpu_sc as plsc

N = 10000
E = 320000
NFEAT = 128
NHID = 128
NCLASS = 64
NHEADS = 8
ALPHA = 0.2

NC = 2
NS = 16
NW = NC * NS
CH = 80
TPH = 16
NP = 10240
RPP = NP // NS
ZR = 128
RB = 400
GRID = N // RB

f32 = jnp.float32


def _mesh():
    return plsc.VectorSubcoreMesh(core_axis_name="c", subcore_axis_name="s",
                                  num_cores=NC, num_subcores=NS)


def _tc1_body(x_ref, w_ref, a1_ref, a2_ref, f1_ref, f2_ref, *wh_refs):
    wh = jnp.dot(x_ref[...], w_ref[...], preferred_element_type=f32)
    f1_ref[...] = jnp.dot(wh, a1_ref[...], preferred_element_type=f32)
    f2_ref[...] = jnp.dot(wh, a2_ref[...], preferred_element_type=f32)
    for h in range(NHEADS):
        wh_refs[h][...] = wh[:, h * NHID:(h + 1) * NHID]


def _tc1(x, w_all, a1p, a2p):
    kH = NHEADS * NHID
    return pl.pallas_call(
        _tc1_body,
        grid=(GRID,),
        in_specs=[
            pl.BlockSpec((RB, NFEAT), lambda i: (i, 0)),
            pl.BlockSpec((NFEAT, kH), lambda i: (0, 0)),
            pl.BlockSpec((kH, TPH), lambda i: (0, 0)),
            pl.BlockSpec((kH, TPH), lambda i: (0, 0)),
        ],
        out_specs=[pl.BlockSpec((RB, TPH), lambda i: (i, 0)),
                   pl.BlockSpec((RB, TPH), lambda i: (i, 0))] +
                  [pl.BlockSpec((RB, NHID), lambda i: (i, 0))
                   for _ in range(NHEADS)],
        out_shape=[jax.ShapeDtypeStruct((N, TPH), f32),
                   jax.ShapeDtypeStruct((N, TPH), f32)] +
                  [jax.ShapeDtypeStruct((N, NHID), f32)
                   for _ in range(NHEADS)],
    )(x, w_all, a1p, a2p)


def _tc2_body(p_ref, w_ref, a1_ref, a2_ref, wh2_ref, f1_ref, f2_ref):
    p = p_ref[...]
    hsum = p[0] + p[1]
    hact = jnp.where(hsum > 0, hsum, jnp.exp(hsum) - 1.0)
    hcat = jnp.transpose(hact, (1, 0, 2)).reshape(RB, NHEADS * NHID)
    wh2 = jnp.dot(hcat, w_ref[...], preferred_element_type=f32)
    wh2_ref[...] = wh2
    f1_ref[...] = jnp.dot(wh2, a1_ref[...], preferred_element_type=f32)
    f2_ref[...] = jnp.dot(wh2, a2_ref[...], preferred_element_type=f32)


def _tc2(p, w_out, a1p, a2p):
    kH = NHEADS * NHID
    return pl.pallas_call(
        _tc2_body,
        grid=(GRID,),
        in_specs=[
            pl.BlockSpec((NC, NHEADS, RB, NHID), lambda i: (0, 0, i, 0)),
            pl.BlockSpec((kH, NCLASS), lambda i: (0, 0)),
            pl.BlockSpec((NCLASS, TPH), lambda i: (0, 0)),
            pl.BlockSpec((NCLASS, TPH), lambda i: (0, 0)),
        ],
        out_specs=[pl.BlockSpec((RB, NCLASS), lambda i: (i, 0)),
                   pl.BlockSpec((RB, TPH), lambda i: (i, 0)),
                   pl.BlockSpec((RB, TPH), lambda i: (i, 0))],
        out_shape=[jax.ShapeDtypeStruct((N, NCLASS), f32),
                   jax.ShapeDtypeStruct((N, TPH), f32),
                   jax.ShapeDtypeStruct((N, TPH), f32)],
    )(p, w_out, a1p, a2p)


def _tc3_body(p_ref, o_ref):
    p = p_ref[...]
    z = p[0] + p[1]
    m = jnp.max(z, axis=1, keepdims=True)
    lse = m + jnp.log(jnp.sum(jnp.exp(z - m), axis=1, keepdims=True))
    o_ref[...] = z - lse


def _tc3(p):
    return pl.pallas_call(
        _tc3_body,
        grid=(GRID,),
        in_specs=[pl.BlockSpec((NC, RB, NCLASS), lambda i: (0, i, 0))],
        out_specs=pl.BlockSpec((RB, NCLASS), lambda i: (i, 0)),
        out_shape=jax.ShapeDtypeStruct((N, NCLASS), f32),
    )(p)


def _tct_body(a_ref, o_ref):
    o_ref[...] = a_ref[...].T


def _tct(att):
    BE = 2560
    return pl.pallas_call(
        _tct_body,
        grid=(E // BE,),
        in_specs=[pl.BlockSpec((BE, TPH), lambda i: (i, 0))],
        out_specs=pl.BlockSpec((TPH, BE), lambda i: (0, i)),
        out_shape=jax.ShapeDtypeStruct((TPH, E), f32),
    )(att)


def _sc_att(row, col, f1t, f2t):
    EPC = E // NC
    EPS1 = E // NS
    EPS2 = EPC // NS
    NCH1 = EPS1 // CH
    NCH2 = EPS2 // CH

    @functools.partial(
        pl.kernel,
        out_type=[jax.ShapeDtypeStruct((E, TPH), f32),
                  jax.ShapeDtypeStruct((NC * NP, TPH), f32)],
        mesh=_mesh(),
        compiler_params=pltpu.CompilerParams(use_tc_tiling_on_sc=False),
        scratch_types=[
            pltpu.VMEM((2, CH), jnp.int32),
            pltpu.VMEM((2, CH), jnp.int32),
            pltpu.VMEM((2, CH), jnp.int32),
            pltpu.VMEM((2, CH, TPH), f32),
            pltpu.VMEM((2, CH, TPH), f32),
            pltpu.VMEM((2, CH, TPH), f32),
            pltpu.VMEM((2, CH, TPH), f32),
            pltpu.VMEM((ZR, TPH), f32),
            pltpu.VMEM_SHARED((NP, TPH), f32),
            pltpu.SemaphoreType.DMA,
            pltpu.SemaphoreType.DMA,
            pltpu.SemaphoreType.DMA,
            pltpu.SemaphoreType.DMA,
        ],
    )
    def k(row_ref, col_ref, f1t_ref, f2t_ref, att_ref, den_ref,
          rowv, colv, rowv2, f1g, f2g, sbuf, dg, zbuf, dshared,
          semL0, semL1, semG0, semG1):
        semL = (semL0, semL1)
        semG = (semG0, semG1)
        c = lax.axis_index("c")
        s = lax.axis_index("s")

        def zb(j, carry):
            zbuf[j, :] = jnp.zeros((TPH,), f32)
            return carry
        lax.fori_loop(0, ZR, zb, 0)
        for i in range(RPP // ZR):
            pltpu.sync_copy(zbuf, dshared.at[pl.ds(s * RPP + i * ZR, ZR)])
        plsc.subcore_barrier()

        def base1(ci):
            return s * EPS1 + ci * CH

        def issue_loads1(ci, b):
            pltpu.async_copy(row_ref.at[pl.ds(base1(ci), CH)], rowv.at[b],
                             semL[b])
            pltpu.async_copy(col_ref.at[pl.ds(base1(ci), CH)], colv.at[b],
                             semL[b])

        def wait_loads1(ci, b):
            pltpu.make_async_copy(row_ref.at[pl.ds(base1(ci), CH)],
                                  rowv.at[b], semL[b]).wait()
            pltpu.make_async_copy(col_ref.at[pl.ds(base1(ci), CH)],
                                  colv.at[b], semL[b]).wait()

        def issue_gathers(b):
            pltpu.async_copy(f1t_ref.at[rowv.at[b]], f1g.at[b], semG[b])
            pltpu.async_copy(f2t_ref.at[colv.at[b]], f2g.at[b], semG[b])

        def wait_gathers(b):
            pltpu.make_async_copy(f1t_ref.at[rowv.at[b]], f1g.at[b],
                                  semG[b]).wait()
            pltpu.make_async_copy(f2t_ref.at[colv.at[b]], f2g.at[b],
                                  semG[b]).wait()

        def compute_s(b):
            def ed(j, carry2):
                v = f1g[b, j, :] + f2g[b, j, :]
                sbuf[b, j, :] = jnp.exp(jnp.maximum(v, ALPHA * v))
                return carry2
            lax.fori_loop(0, CH, ed, 0, unroll=2)

        def halfstep1(ci, b, more_loads):
            nb = 1 - b
            wait_loads1(ci + 1, nb)
            issue_gathers(nb)
            wait_gathers(b)
            compute_s(b)
            pltpu.sync_copy(sbuf.at[b], dshared.at[rowv.at[b]], add=True)

            @pl.when(more_loads)
            def _():
                issue_loads1(ci + 2, b)

        issue_loads1(0, 0)
        issue_loads1(1, 1)
        wait_loads1(0, 0)
        issue_gathers(0)

        def pair1(pk, carry):
            ci0 = 2 * pk
            halfstep1(ci0, 0, ci0 + 2 < NCH1)
            @pl.when(ci0 + 2 < NCH1)
            def _():
                halfstep1(ci0 + 1, 1, ci0 + 3 < NCH1)
            return carry
        lax.fori_loop(0, NCH1 // 2, pair1, 0)
        wait_gathers(1)
        compute_s(1)
        pltpu.sync_copy(sbuf.at[1], dshared.at[rowv.at[1]], add=True)
        plsc.subcore_barrier()

        pltpu.sync_copy(dshared.at[pl.ds(s * RPP, RPP)],
                        den_ref.at[pl.ds(c * NP + s * RPP, RPP)])
        plsc.subcore_barrier()

        def base2(ci):
            return c * EPC + s * EPS2 + ci * CH

        def issue_loads2(ci, b):
            pltpu.async_copy(row_ref.at[pl.ds(base2(ci), CH)], rowv.at[b],
                             semL[b])
            pltpu.async_copy(col_ref.at[pl.ds(base2(ci), CH)], colv.at[b],
                             semL[b])

        def wait_loads2(ci, b):
            pltpu.make_async_copy(row_ref.at[pl.ds(base2(ci), CH)],
                                  rowv.at[b], semL[b]).wait()
            pltpu.make_async_copy(col_ref.at[pl.ds(base2(ci), CH)],
                                  colv.at[b], semL[b]).wait()

        def issue_gathers2(b):
            def off(j, carry2):
                rowv2[b, pl.ds(j * 16, 16)] = (rowv[b, pl.ds(j * 16, 16)]
                                               + c * NP)
                return carry2
            lax.fori_loop(0, CH // 16, off, 0)
            pltpu.async_copy(f1t_ref.at[rowv.at[b]], f1g.at[b], semG[b])
            pltpu.async_copy(f2t_ref.at[colv.at[b]], f2g.at[b], semG[b])
            pltpu.async_copy(den_ref.at[rowv2.at[b]], dg.at[b], semG[b])

        def wait_gathers2(b):
            wait_gathers(b)
            pltpu.make_async_copy(den_ref.at[rowv2.at[b]], dg.at[b],
                                  semG[b]).wait()

        def compute_att(b):
            def ed(j, carry2):
                v = f1g[b, j, :] + f2g[b, j, :]
                sval = jnp.exp(jnp.maximum(v, ALPHA * v))
                sbuf[b, j, :] = sval / (dg[b, j, :] + 1e-10)
                return carry2
            lax.fori_loop(0, CH, ed, 0, unroll=2)

        def halfstep2(ci, b, more_loads):
            nb = 1 - b
            wait_loads2(ci + 1, nb)
            issue_gathers2(nb)
            wait_gathers2(b)
            compute_att(b)
            pltpu.sync_copy(sbuf.at[b], att_ref.at[pl.ds(base2(ci), CH)])

            @pl.when(more_loads)
            def _():
                issue_loads2(ci + 2, b)

        issue_loads2(0, 0)
        issue_loads2(1, 1)
        wait_loads2(0, 0)
        issue_gathers2(0)

        def pair2(pk, carry):
            ci0 = 2 * pk
            halfstep2(ci0, 0, ci0 + 2 < NCH2)
            halfstep2(ci0 + 1, 1, ci0 + 3 < NCH2)
            return carry
        lax.fori_loop(0, (NCH2 - 1) // 2, pair2, 0)
        wait_gathers2(0)
        compute_att(0)
        pltpu.sync_copy(sbuf.at[0], att_ref.at[pl.ds(base2(NCH2 - 1), CH)])

    return k(row, col, f1t, f2t)


def _sc_spmm(row, col, att, wh_tables, d):
    nheads = len(wh_tables)
    EPW = E // NW
    CH2 = 80
    NST = CH2 // CH
    NCH = EPW // CH2
    nv = d // 16

    @functools.partial(
        pl.kernel,
        out_type=jax.ShapeDtypeStruct((NC, nheads, NP, d), f32),
        mesh=_mesh(),
        compiler_params=pltpu.CompilerParams(use_tc_tiling_on_sc=False),
        scratch_types=[
            pltpu.VMEM((2 * NST, CH), jnp.int32),
            pltpu.VMEM((2 * NST, CH), jnp.int32),
            pltpu.VMEM((2, CH2), f32),
            pltpu.VMEM((2 * CH2, d), f32),
            pltpu.VMEM((64, d), f32),
            pltpu.VMEM_SHARED((NP, d), f32),
            pltpu.SemaphoreType.DMA,
            pltpu.SemaphoreType.DMA,
            pltpu.SemaphoreType.DMA,
            pltpu.SemaphoreType.DMA,
            pltpu.SemaphoreType.DMA,
            pltpu.SemaphoreType.DMA,
        ],
    )
    def k(row_ref, col_ref, att_ref, *rest):
        wh_refs = rest[:nheads]
        out_ref = rest[nheads]
        (rowv, colv, attv, g, zbuf, ashared,
         semL0, semL1, semG0, semG1, semS0, semS1) = rest[nheads + 1:]
        semL = (semL0, semL1)
        semG = (semG0, semG1)
        semS = (semS0, semS1)
        c = lax.axis_index("c")
        s = lax.axis_index("s")
        wid = c * NS + s
        ebase = wid * EPW

        def zb(j, carry):
            for v in range(nv):
                zbuf[j, pl.ds(v * 16, 16)] = jnp.zeros((16,), f32)
            return carry
        lax.fori_loop(0, 64, zb, 0)

        def mk_issue_loads(h):
            def issue_loads(ci, b):
                base = ebase + ci * CH2
                rbase = base // CH
                pltpu.async_copy(row_ref.at[pl.ds(rbase, NST)],
                                 rowv.at[pl.ds(NST * b, NST)], semL[b])
                pltpu.async_copy(col_ref.at[pl.ds(rbase, NST)],
                                 colv.at[pl.ds(NST * b, NST)], semL[b])
                pltpu.async_copy(att_ref.at[h, pl.ds(base, CH2)],
                                 attv.at[b], semL[b])
            return issue_loads

        def mk_wait_loads(h):
            def wait_loads(ci, b):
                base = ebase + ci * CH2
                rbase = base // CH
                pltpu.make_async_copy(row_ref.at[pl.ds(rbase, NST)],
                                      rowv.at[pl.ds(NST * b, NST)],
                                      semL[b]).wait()
                pltpu.make_async_copy(col_ref.at[pl.ds(rbase, NST)],
                                      colv.at[pl.ds(NST * b, NST)],
                                      semL[b]).wait()
                pltpu.make_async_copy(att_ref.at[h, pl.ds(base, CH2)],
                                      attv.at[b], semL[b]).wait()
            return wait_loads

        for h in range(nheads):
            for i in range(RPP // 64):
                pltpu.sync_copy(zbuf, ashared.at[pl.ds(s * RPP + i * 64, 64)])
            plsc.subcore_barrier()

            wh_ref = wh_refs[h]
            issue_loads = mk_issue_loads(h)
            wait_loads = mk_wait_loads(h)

            def issue_gather(b):
                for st in range(NST):
                    pltpu.async_copy(wh_ref.at[colv.at[NST * b + st]],
                                     g.at[pl.ds(b * CH2 + st * CH, CH)],
                                     semG[b])

            def wait_gather(b):
                for st in range(NST):
                    pltpu.make_async_copy(wh_ref.at[colv.at[NST * b + st]],
                                          g.at[pl.ds(b * CH2 + st * CH, CH)],
                                          semG[b]).wait()

            def issue_scatter(b):
                for st in range(NST):
                    pltpu.async_copy(g.at[pl.ds(b * CH2 + st * CH, CH)],
                                     ashared.at[rowv.at[NST * b + st]],
                                     semS[b], add=True)

            def drain_scatter(b):
                for st in range(NST):
                    pltpu.make_async_copy(g.at[pl.ds(b * CH2 + st * CH, CH)],
                                          ashared.at[rowv.at[NST * b + st]],
                                          semS[b]).wait()

            def scale(b):
                def grp(jg, carry2):
                    ag = attv[b, pl.ds(jg * 16, 16)]
                    for l in range(16):
                        a = ag[l]
                        for v in range(nv):
                            g[b * CH2 + jg * 16 + l, pl.ds(v * 16, 16)] = (
                                g[b * CH2 + jg * 16 + l, pl.ds(v * 16, 16)]
                                * a)
                    return carry2
                lax.fori_loop(0, CH2 // 16, grp, 0)

            def halfstep(ci, b, more_loads):
                nb = 1 - b
                wait_gather(b)
                scale(b)
                wait_loads(ci + 1, nb)

                @pl.when(ci >= 1)
                def _():
                    drain_scatter(nb)
                issue_gather(nb)
                issue_scatter(b)

                @pl.when(more_loads)
                def _():
                    issue_loads(ci + 2, b)

            issue_loads(0, 0)
            issue_loads(1, 1)
            wait_loads(0, 0)
            issue_gather(0)

            def pair(pk, carry):
                ci0 = 2 * pk
                halfstep(ci0, 0, ci0 + 2 < NCH)
                halfstep(ci0 + 1, 1, ci0 + 3 < NCH)
                return carry
            lax.fori_loop(0, NCH // 2, pair, 0)

            wait_gather(0)
            scale(0)
            drain_scatter(1)
            issue_scatter(0)
            drain_scatter(0)
            plsc.subcore_barrier()

            pltpu.sync_copy(ashared.at[pl.ds(s * RPP, RPP)],
                            out_ref.at[c, h, pl.ds(s * RPP, RPP)])
            plsc.subcore_barrier()

    return k(row, col, att, *wh_tables)


def kernel(x, adj, W_heads, a1_heads, a2_heads, W_out, a1_out, a2_out):
    kH = NHEADS * NHID
    row = adj[0]
    col = adj[1]

    w_all = jnp.transpose(W_heads, (1, 0, 2)).reshape(NFEAT, kH)
    eye = jnp.eye(NHEADS, dtype=f32)
    a1p = (a1_heads[:, :, None] * eye[:, None, :]).reshape(kH, NHEADS)
    a1p = jnp.pad(a1p, ((0, 0), (0, TPH - NHEADS)))
    a2p = (a2_heads[:, :, None] * eye[:, None, :]).reshape(kH, NHEADS)
    a2p = jnp.pad(a2p, ((0, 0), (0, TPH - NHEADS)))
    a1op = jnp.pad(a1_out[:, None], ((0, 0), (0, TPH - 1)))
    a2op = jnp.pad(a2_out[:, None], ((0, 0), (0, TPH - 1)))

    tc1 = _tc1(x, w_all, a1p, a2p)
    f1t, f2t = tc1[0], tc1[1]
    whs = tc1[2:]
    att1, _ = _sc_att(row, col, f1t, f2t)
    att1t = _tct(att1)
    row2 = row.reshape(E // CH, CH)
    col2 = col.reshape(E // CH, CH)
    p1 = _sc_spmm(row2, col2, att1t, list(whs), NHID)

    wh2, f1t2, f2t2 = _tc2(p1, W_out, a1op, a2op)
    att2, _ = _sc_att(row, col, f1t2, f2t2)
    att2t = _tct(att2)
    p2 = _sc_spmm(row2, col2, att2t, [wh2], NCLASS)

    return _tc3(p2[:, 0])


# speedup vs baseline: 1.0746x; 1.0746x over previous
import functools
import jax
import jax.numpy as jnp
from jax import lax
from jax.experimental import pallas as pl
from jax.experimental.pallas import tpu as pltpu
from jax.experimental.pallas import tpu_sc as plsc

N = 10000
E = 320000
NFEAT = 128
NHID = 128
NCLASS = 64
NHEADS = 8
ALPHA = 0.2

NC = 2
NS = 16
NW = NC * NS
CH = 80
TPH = 16
NP = 10240
RPP = NP // NS
ZR = 128
RB = 400
GRID = N // RB

f32 = jnp.float32


def _mesh():
    return plsc.VectorSubcoreMesh(core_axis_name="c", subcore_axis_name="s",
                                  num_cores=NC, num_subcores=NS)


def _tc1_body(x_ref, w_ref, a1_ref, a2_ref, f1_ref, f2_ref, *wh_refs):
    wh = jnp.dot(x_ref[...], w_ref[...], preferred_element_type=f32)
    f1_ref[...] = jnp.dot(wh, a1_ref[...], preferred_element_type=f32)
    f2_ref[...] = jnp.dot(wh, a2_ref[...], preferred_element_type=f32)
    for h in range(NHEADS):
        wh_refs[h][...] = wh[:, h * NHID:(h + 1) * NHID]


def _tc1(x, w_all, a1p, a2p):
    kH = NHEADS * NHID
    return pl.pallas_call(
        _tc1_body,
        grid=(GRID,),
        in_specs=[
            pl.BlockSpec((RB, NFEAT), lambda i: (i, 0)),
            pl.BlockSpec((NFEAT, kH), lambda i: (0, 0)),
            pl.BlockSpec((kH, TPH), lambda i: (0, 0)),
            pl.BlockSpec((kH, TPH), lambda i: (0, 0)),
        ],
        out_specs=[pl.BlockSpec((RB, TPH), lambda i: (i, 0)),
                   pl.BlockSpec((RB, TPH), lambda i: (i, 0))] +
                  [pl.BlockSpec((RB, NHID), lambda i: (i, 0))
                   for _ in range(NHEADS)],
        out_shape=[jax.ShapeDtypeStruct((N, TPH), f32),
                   jax.ShapeDtypeStruct((N, TPH), f32)] +
                  [jax.ShapeDtypeStruct((N, NHID), f32)
                   for _ in range(NHEADS)],
    )(x, w_all, a1p, a2p)


def _tc2_body(p_ref, w_ref, a1_ref, a2_ref, wh2_ref, f1_ref, f2_ref):
    p = p_ref[...]
    hsum = p[0] + p[1]
    hact = jnp.where(hsum > 0, hsum, jnp.exp(hsum) - 1.0)
    hcat = jnp.transpose(hact, (1, 0, 2)).reshape(RB, NHEADS * NHID)
    wh2 = jnp.dot(hcat, w_ref[...], preferred_element_type=f32)
    wh2_ref[...] = wh2
    f1_ref[...] = jnp.dot(wh2, a1_ref[...], preferred_element_type=f32)
    f2_ref[...] = jnp.dot(wh2, a2_ref[...], preferred_element_type=f32)


def _tc2(p, w_out, a1p, a2p):
    kH = NHEADS * NHID
    return pl.pallas_call(
        _tc2_body,
        grid=(GRID,),
        in_specs=[
            pl.BlockSpec((NC, NHEADS, RB, NHID), lambda i: (0, 0, i, 0)),
            pl.BlockSpec((kH, NCLASS), lambda i: (0, 0)),
            pl.BlockSpec((NCLASS, TPH), lambda i: (0, 0)),
            pl.BlockSpec((NCLASS, TPH), lambda i: (0, 0)),
        ],
        out_specs=[pl.BlockSpec((RB, NCLASS), lambda i: (i, 0)),
                   pl.BlockSpec((RB, TPH), lambda i: (i, 0)),
                   pl.BlockSpec((RB, TPH), lambda i: (i, 0))],
        out_shape=[jax.ShapeDtypeStruct((N, NCLASS), f32),
                   jax.ShapeDtypeStruct((N, TPH), f32),
                   jax.ShapeDtypeStruct((N, TPH), f32)],
    )(p, w_out, a1p, a2p)


def _tc3_body(p_ref, o_ref):
    p = p_ref[...]
    z = p[0] + p[1]
    m = jnp.max(z, axis=1, keepdims=True)
    lse = m + jnp.log(jnp.sum(jnp.exp(z - m), axis=1, keepdims=True))
    o_ref[...] = z - lse


def _tc3(p):
    return pl.pallas_call(
        _tc3_body,
        grid=(GRID,),
        in_specs=[pl.BlockSpec((NC, RB, NCLASS), lambda i: (0, i, 0))],
        out_specs=pl.BlockSpec((RB, NCLASS), lambda i: (i, 0)),
        out_shape=jax.ShapeDtypeStruct((N, NCLASS), f32),
    )(p)


def _tct_body(a_ref, o_ref):
    o_ref[...] = a_ref[...].T


def _tct(att):
    BE = 2560
    return pl.pallas_call(
        _tct_body,
        grid=(E // BE,),
        in_specs=[pl.BlockSpec((BE, TPH), lambda i: (i, 0))],
        out_specs=pl.BlockSpec((TPH, BE), lambda i: (0, i)),
        out_shape=jax.ShapeDtypeStruct((TPH, E), f32),
    )(att)


def _sc_att(row, col, f1t, f2t):
    EPC = E // NC
    EPS1 = E // NS
    EPS2 = EPC // NS
    NCH1 = EPS1 // CH
    NCH2 = EPS2 // CH

    @functools.partial(
        pl.kernel,
        out_type=[jax.ShapeDtypeStruct((E, TPH), f32),
                  jax.ShapeDtypeStruct((NC * NP, TPH), f32)],
        mesh=_mesh(),
        compiler_params=pltpu.CompilerParams(use_tc_tiling_on_sc=False),
        scratch_types=[
            pltpu.VMEM((2, CH), jnp.int32),
            pltpu.VMEM((2, CH), jnp.int32),
            pltpu.VMEM((2, CH), jnp.int32),
            pltpu.VMEM((2, CH, TPH), f32),
            pltpu.VMEM((2, CH, TPH), f32),
            pltpu.VMEM((2, CH, TPH), f32),
            pltpu.VMEM((2, CH, TPH), f32),
            pltpu.VMEM((ZR, TPH), f32),
            pltpu.VMEM_SHARED((NP, TPH), f32),
            pltpu.SemaphoreType.DMA,
            pltpu.SemaphoreType.DMA,
            pltpu.SemaphoreType.DMA,
            pltpu.SemaphoreType.DMA,
            pltpu.SemaphoreType.DMA,
            pltpu.SemaphoreType.DMA,
        ],
    )
    def k(row_ref, col_ref, f1t_ref, f2t_ref, att_ref, den_ref,
          rowv, colv, rowv2, f1g, f2g, sbuf, dg, zbuf, dshared,
          semL0, semL1, semG0, semG1, semS0, semS1):
        semL = (semL0, semL1)
        semG = (semG0, semG1)
        semS = (semS0, semS1)
        c = lax.axis_index("c")
        s = lax.axis_index("s")

        def zb(j, carry):
            zbuf[j, :] = jnp.zeros((TPH,), f32)
            return carry
        lax.fori_loop(0, ZR, zb, 0)
        for i in range(RPP // ZR):
            pltpu.sync_copy(zbuf, dshared.at[pl.ds(s * RPP + i * ZR, ZR)])
        plsc.subcore_barrier()

        def base1(ci):
            return s * EPS1 + ci * CH

        def issue_loads1(ci, b):
            pltpu.async_copy(row_ref.at[pl.ds(base1(ci), CH)], rowv.at[b],
                             semL[b])
            pltpu.async_copy(col_ref.at[pl.ds(base1(ci), CH)], colv.at[b],
                             semL[b])

        def wait_loads1(ci, b):
            pltpu.make_async_copy(row_ref.at[pl.ds(base1(ci), CH)],
                                  rowv.at[b], semL[b]).wait()
            pltpu.make_async_copy(col_ref.at[pl.ds(base1(ci), CH)],
                                  colv.at[b], semL[b]).wait()

        def issue_gathers(b):
            pltpu.async_copy(f1t_ref.at[rowv.at[b]], f1g.at[b], semG[b])
            pltpu.async_copy(f2t_ref.at[colv.at[b]], f2g.at[b], semG[b])

        def wait_gathers(b):
            pltpu.make_async_copy(f1t_ref.at[rowv.at[b]], f1g.at[b],
                                  semG[b]).wait()
            pltpu.make_async_copy(f2t_ref.at[colv.at[b]], f2g.at[b],
                                  semG[b]).wait()

        def compute_s(b):
            def ed(j, carry2):
                v = f1g[b, j, :] + f2g[b, j, :]
                sbuf[b, j, :] = jnp.exp(jnp.maximum(v, ALPHA * v))
                return carry2
            lax.fori_loop(0, CH, ed, 0, unroll=2)

        def halfstep1(ci, b, more_loads):
            nb = 1 - b
            wait_loads1(ci + 1, nb)
            issue_gathers(nb)
            wait_gathers(b)

            @pl.when(ci >= 2)
            def _():
                pltpu.make_async_copy(sbuf.at[b], dshared.at[rowv.at[b]],
                                      semS[b]).wait()
            compute_s(b)
            pltpu.async_copy(sbuf.at[b], dshared.at[rowv.at[b]], semS[b],
                             add=True)

            @pl.when(more_loads)
            def _():
                issue_loads1(ci + 2, b)

        issue_loads1(0, 0)
        issue_loads1(1, 1)
        wait_loads1(0, 0)
        issue_gathers(0)

        def pair1(pk, carry):
            ci0 = 2 * pk
            halfstep1(ci0, 0, ci0 + 2 < NCH1)
            @pl.when(ci0 + 2 < NCH1)
            def _():
                halfstep1(ci0 + 1, 1, ci0 + 3 < NCH1)
            return carry
        lax.fori_loop(0, NCH1 // 2, pair1, 0)
        wait_gathers(1)
        pltpu.make_async_copy(sbuf.at[1], dshared.at[rowv.at[1]],
                              semS[1]).wait()
        compute_s(1)
        pltpu.sync_copy(sbuf.at[1], dshared.at[rowv.at[1]], add=True)
        pltpu.make_async_copy(sbuf.at[0], dshared.at[rowv.at[0]],
                              semS[0]).wait()
        plsc.subcore_barrier()

        pltpu.sync_copy(dshared.at[pl.ds(s * RPP, RPP)],
                        den_ref.at[pl.ds(c * NP + s * RPP, RPP)])
        plsc.subcore_barrier()

        def base2(ci):
            return c * EPC + s * EPS2 + ci * CH

        def issue_loads2(ci, b):
            pltpu.async_copy(row_ref.at[pl.ds(base2(ci), CH)], rowv.at[b],
                             semL[b])
            pltpu.async_copy(col_ref.at[pl.ds(base2(ci), CH)], colv.at[b],
                             semL[b])

        def wait_loads2(ci, b):
            pltpu.make_async_copy(row_ref.at[pl.ds(base2(ci), CH)],
                                  rowv.at[b], semL[b]).wait()
            pltpu.make_async_copy(col_ref.at[pl.ds(base2(ci), CH)],
                                  colv.at[b], semL[b]).wait()

        def issue_gathers2(b):
            def off(j, carry2):
                rowv2[b, pl.ds(j * 16, 16)] = (rowv[b, pl.ds(j * 16, 16)]
                                               + c * NP)
                return carry2
            lax.fori_loop(0, CH // 16, off, 0)
            pltpu.async_copy(f1t_ref.at[rowv.at[b]], f1g.at[b], semG[b])
            pltpu.async_copy(f2t_ref.at[colv.at[b]], f2g.at[b], semG[b])
            pltpu.async_copy(den_ref.at[rowv2.at[b]], dg.at[b], semG[b])

        def wait_gathers2(b):
            wait_gathers(b)
            pltpu.make_async_copy(den_ref.at[rowv2.at[b]], dg.at[b],
                                  semG[b]).wait()

        def compute_att(b):
            def ed(j, carry2):
                v = f1g[b, j, :] + f2g[b, j, :]
                sval = jnp.exp(jnp.maximum(v, ALPHA * v))
                sbuf[b, j, :] = sval / (dg[b, j, :] + 1e-10)
                return carry2
            lax.fori_loop(0, CH, ed, 0, unroll=2)

        def halfstep2(ci, b, more_loads):
            nb = 1 - b
            wait_loads2(ci + 1, nb)
            issue_gathers2(nb)
            wait_gathers2(b)

            @pl.when(ci >= 2)
            def _():
                pltpu.make_async_copy(
                    sbuf.at[b], att_ref.at[pl.ds(base2(ci), CH)],
                    semS[b]).wait()
            compute_att(b)
            pltpu.async_copy(sbuf.at[b], att_ref.at[pl.ds(base2(ci), CH)],
                             semS[b])

            @pl.when(more_loads)
            def _():
                issue_loads2(ci + 2, b)

        issue_loads2(0, 0)
        issue_loads2(1, 1)
        wait_loads2(0, 0)
        issue_gathers2(0)

        def pair2(pk, carry):
            ci0 = 2 * pk
            halfstep2(ci0, 0, ci0 + 2 < NCH2)
            halfstep2(ci0 + 1, 1, ci0 + 3 < NCH2)
            return carry
        lax.fori_loop(0, (NCH2 - 1) // 2, pair2, 0)
        wait_gathers2(0)
        pltpu.make_async_copy(
            sbuf.at[0], att_ref.at[pl.ds(base2(NCH2 - 1), CH)],
            semS[0]).wait()
        compute_att(0)
        pltpu.sync_copy(sbuf.at[0], att_ref.at[pl.ds(base2(NCH2 - 1), CH)])
        pltpu.make_async_copy(
            sbuf.at[1], att_ref.at[pl.ds(base2(NCH2 - 1), CH)],
            semS[1]).wait()

    return k(row, col, f1t, f2t)


def _sc_spmm(row, col, att, wh_tables, d):
    nheads = len(wh_tables)
    EPW = E // NW
    CH2 = 80
    NST = CH2 // CH
    NCH = EPW // CH2
    nv = d // 16

    @functools.partial(
        pl.kernel,
        out_type=jax.ShapeDtypeStruct((NC, nheads, NP, d), f32),
        mesh=_mesh(),
        compiler_params=pltpu.CompilerParams(use_tc_tiling_on_sc=False),
        scratch_types=[
            pltpu.VMEM((2 * NST, CH), jnp.int32),
            pltpu.VMEM((2 * NST, CH), jnp.int32),
            pltpu.VMEM((2, CH2), f32),
            pltpu.VMEM((2 * CH2, d), f32),
            pltpu.VMEM((64, d), f32),
            pltpu.VMEM_SHARED((NP, d), f32),
            pltpu.SemaphoreType.DMA,
            pltpu.SemaphoreType.DMA,
            pltpu.SemaphoreType.DMA,
            pltpu.SemaphoreType.DMA,
            pltpu.SemaphoreType.DMA,
            pltpu.SemaphoreType.DMA,
        ],
    )
    def k(row_ref, col_ref, att_ref, *rest):
        wh_refs = rest[:nheads]
        out_ref = rest[nheads]
        (rowv, colv, attv, g, zbuf, ashared,
         semL0, semL1, semG0, semG1, semS0, semS1) = rest[nheads + 1:]
        semL = (semL0, semL1)
        semG = (semG0, semG1)
        semS = (semS0, semS1)
        c = lax.axis_index("c")
        s = lax.axis_index("s")
        wid = c * NS + s
        ebase = wid * EPW

        def zb(j, carry):
            for v in range(nv):
                zbuf[j, pl.ds(v * 16, 16)] = jnp.zeros((16,), f32)
            return carry
        lax.fori_loop(0, 64, zb, 0)

        def mk_issue_loads(h):
            def issue_loads(ci, b):
                base = ebase + ci * CH2
                rbase = base // CH
                pltpu.async_copy(row_ref.at[pl.ds(rbase, NST)],
                                 rowv.at[pl.ds(NST * b, NST)], semL[b])
                pltpu.async_copy(col_ref.at[pl.ds(rbase, NST)],
                                 colv.at[pl.ds(NST * b, NST)], semL[b])
                pltpu.async_copy(att_ref.at[h, pl.ds(base, CH2)],
                                 attv.at[b], semL[b])
            return issue_loads

        def mk_wait_loads(h):
            def wait_loads(ci, b):
                base = ebase + ci * CH2
                rbase = base // CH
                pltpu.make_async_copy(row_ref.at[pl.ds(rbase, NST)],
                                      rowv.at[pl.ds(NST * b, NST)],
                                      semL[b]).wait()
                pltpu.make_async_copy(col_ref.at[pl.ds(rbase, NST)],
                                      colv.at[pl.ds(NST * b, NST)],
                                      semL[b]).wait()
                pltpu.make_async_copy(att_ref.at[h, pl.ds(base, CH2)],
                                      attv.at[b], semL[b]).wait()
            return wait_loads

        for h in range(nheads):
            for i in range(RPP // 64):
                pltpu.sync_copy(zbuf, ashared.at[pl.ds(s * RPP + i * 64, 64)])
            plsc.subcore_barrier()

            wh_ref = wh_refs[h]
            issue_loads = mk_issue_loads(h)
            wait_loads = mk_wait_loads(h)

            def issue_gather(b):
                for st in range(NST):
                    pltpu.async_copy(wh_ref.at[colv.at[NST * b + st]],
                                     g.at[pl.ds(b * CH2 + st * CH, CH)],
                                     semG[b])

            def wait_gather(b):
                for st in range(NST):
                    pltpu.make_async_copy(wh_ref.at[colv.at[NST * b + st]],
                                          g.at[pl.ds(b * CH2 + st * CH, CH)],
                                          semG[b]).wait()

            def issue_scatter(b):
                for st in range(NST):
                    pltpu.async_copy(g.at[pl.ds(b * CH2 + st * CH, CH)],
                                     ashared.at[rowv.at[NST * b + st]],
                                     semS[b], add=True)

            def drain_scatter(b):
                for st in range(NST):
                    pltpu.make_async_copy(g.at[pl.ds(b * CH2 + st * CH, CH)],
                                          ashared.at[rowv.at[NST * b + st]],
                                          semS[b]).wait()

            def scale(b):
                def grp(jg, carry2):
                    ag = attv[b, pl.ds(jg * 16, 16)]
                    for l in range(16):
                        a = ag[l]
                        for v in range(nv):
                            g[b * CH2 + jg * 16 + l, pl.ds(v * 16, 16)] = (
                                g[b * CH2 + jg * 16 + l, pl.ds(v * 16, 16)]
                                * a)
                    return carry2
                lax.fori_loop(0, CH2 // 16, grp, 0)

            def halfstep(ci, b, more_loads):
                nb = 1 - b
                wait_loads(ci + 1, nb)
                issue_gather(nb)
                wait_gather(b)
                scale(b)
                issue_scatter(b)
                drain_scatter(b)

                @pl.when(more_loads)
                def _():
                    issue_loads(ci + 2, b)

            issue_loads(0, 0)
            issue_loads(1, 1)
            wait_loads(0, 0)
            issue_gather(0)

            def pair(pk, carry):
                ci0 = 2 * pk
                halfstep(ci0, 0, ci0 + 2 < NCH)
                halfstep(ci0 + 1, 1, ci0 + 3 < NCH)
                return carry
            lax.fori_loop(0, NCH // 2, pair, 0)

            wait_gather(0)
            scale(0)
            issue_scatter(0)
            drain_scatter(0)
            plsc.subcore_barrier()

            pltpu.sync_copy(ashared.at[pl.ds(s * RPP, RPP)],
                            out_ref.at[c, h, pl.ds(s * RPP, RPP)])
            plsc.subcore_barrier()

    return k(row, col, att, *wh_tables)


def kernel(x, adj, W_heads, a1_heads, a2_heads, W_out, a1_out, a2_out):
    kH = NHEADS * NHID
    row = adj[0]
    col = adj[1]

    w_all = jnp.transpose(W_heads, (1, 0, 2)).reshape(NFEAT, kH)
    eye = jnp.eye(NHEADS, dtype=f32)
    a1p = (a1_heads[:, :, None] * eye[:, None, :]).reshape(kH, NHEADS)
    a1p = jnp.pad(a1p, ((0, 0), (0, TPH - NHEADS)))
    a2p = (a2_heads[:, :, None] * eye[:, None, :]).reshape(kH, NHEADS)
    a2p = jnp.pad(a2p, ((0, 0), (0, TPH - NHEADS)))
    a1op = jnp.pad(a1_out[:, None], ((0, 0), (0, TPH - 1)))
    a2op = jnp.pad(a2_out[:, None], ((0, 0), (0, TPH - 1)))

    tc1 = _tc1(x, w_all, a1p, a2p)
    f1t, f2t = tc1[0], tc1[1]
    whs = tc1[2:]
    att1, _ = _sc_att(row, col, f1t, f2t)
    att1t = _tct(att1)
    row2 = row.reshape(E // CH, CH)
    col2 = col.reshape(E // CH, CH)
    p1 = _sc_spmm(row2, col2, att1t, list(whs), NHID)

    wh2, f1t2, f2t2 = _tc2(p1, W_out, a1op, a2op)
    att2, _ = _sc_att(row, col, f1t2, f2t2)
    att2t = _tct(att2)
    p2 = _sc_spmm(row2, col2, att2t, [wh2], NCLASS)

    return _tc3(p2[:, 0])
